# Initial kernel scaffold; baseline (speedup 1.0000x reference)
import jax
import jax.numpy as jnp
from jax.experimental import pallas as pl


def kernel(table, fc_w, fc_b, evtq_s_ids, evtq_s_lengths, evtq_p_ids, evtq_p_lengths, evtq_o_ids, evtq_o_lengths, evtk_s_ids, evtk_s_lengths, evtk_p_ids, evtk_p_lengths, evtk_o_ids, evtk_o_lengths):
    raise NotImplementedError("write your pallas kernel here")


# speedup vs baseline: 1.4590x; 1.4590x over previous
import functools

import jax
import jax.numpy as jnp
from jax import lax
from jax.experimental import pallas as pl
from jax.experimental.pallas import tpu as pltpu
from jax.experimental.pallas import tpu_sc as plsc

D = 128
B = 4096
L = 20
NSEQ = 6
R = NSEQ * B
NC = 2
NS = 16
NW = NC * NS
RW = R // NW
G = 6
NIDX = G * L
STEPS = RW // G
NDBLK = D // 16


def _pool_body(table_hbm, ids_hbm, coef_hbm, out_hbm,
               ids_v, coef_v, rows0, rows1, ov0, ov1, g0, g1, o0, o1):
    wid = lax.axis_index("s") * NC + lax.axis_index("c")
    base_row = wid * RW

    pltpu.sync_copy(ids_hbm.at[pl.ds(base_row * L, RW * L)], ids_v)
    pltpu.sync_copy(coef_hbm.at[pl.ds(base_row, RW)], coef_v.at[pl.ds(0, RW)])

    def start_gather(s, rows_buf, sem):
        idx = ids_v.at[pl.ds(s * NIDX, NIDX)]
        pltpu.async_copy(table_hbm.at[idx], rows_buf, sem)

    def wait_gather(rows_buf, sem):
        pltpu.make_async_copy(table_hbm.at[pl.ds(0, NIDX)], rows_buf, sem).wait()

    def start_out(s, out_buf, sem):
        pltpu.async_copy(
            out_buf, out_hbm.at[pl.ds((base_row + s * G) * D, G * D)], sem)

    def wait_out(out_buf, sem):
        pltpu.make_async_copy(out_buf, out_hbm.at[pl.ds(0, G * D)], sem).wait()

    def compute(s, rows_buf, out_buf):
        cvec = coef_v[pl.ds(s * G, 16)]
        for i in range(G):
            c = cvec[i]
            for dblk in range(NDBLK):
                sl = pl.ds(dblk * 16, 16)
                e0 = rows_buf[i * L, sl]
                acc = e0
                for j in range(1, L):
                    acc = acc + rows_buf[i * L + j, sl]
                out_buf[pl.ds(i * D + dblk * 16, 16)] = acc * (1.0 / L) + c * e0

    start_gather(0, rows0, g0)
    start_gather(1, rows1, g1)

    def body(so, carry):
        for b, (rows_buf, gsem, out_buf, osem) in enumerate(
                ((rows0, g0, ov0, o0), (rows1, g1, ov1, o1))):
            s = so * 2 + b
            wait_gather(rows_buf, gsem)

            @pl.when(so >= 1)
            def _():
                wait_out(out_buf, osem)

            compute(s, rows_buf, out_buf)
            start_out(s, out_buf, osem)

            @pl.when(s + 2 < STEPS)
            def _():
                start_gather(s + 2, rows_buf, gsem)
        return carry

    lax.fori_loop(0, STEPS // 2, body, 0)
    wait_out(ov0, o0)
    wait_out(ov1, o1)


@functools.partial(
    pl.kernel,
    mesh=plsc.VectorSubcoreMesh(core_axis_name="c", subcore_axis_name="s"),
    out_type=jax.ShapeDtypeStruct((R * D,), jnp.float32),
    scratch_types=[
        pltpu.VMEM((RW * L,), jnp.int32),
        pltpu.VMEM((RW + 16,), jnp.float32),
        pltpu.VMEM((NIDX, D), jnp.float32),
        pltpu.VMEM((NIDX, D), jnp.float32),
        pltpu.VMEM((G * D,), jnp.float32),
        pltpu.VMEM((G * D,), jnp.float32),
        pltpu.SemaphoreType.DMA,
        pltpu.SemaphoreType.DMA,
        pltpu.SemaphoreType.DMA,
        pltpu.SemaphoreType.DMA,
    ],
)
def _pool(table_hbm, ids_hbm, coef_hbm, out_hbm, *rest):
    _pool_body(table_hbm, ids_hbm, coef_hbm, out_hbm, *rest)


def _fc_body(x_ref, w_ref, b_ref, o_ref):
    w = w_ref[...]
    acc = b_ref[0][None, :].astype(jnp.float32)
    for p in range(3):
        acc = acc + jax.lax.dot_general(
            x_ref[0, p], w[p * D:(p + 1) * D, :],
            (((1,), (0,)), ((), ())),
            preferred_element_type=jnp.float32,
            precision=jax.lax.Precision.HIGHEST,
        )
    o_ref[0] = acc


_RB = 512

_fc = pl.pallas_call(
    _fc_body,
    grid=(2, B // _RB),
    in_specs=[
        pl.BlockSpec((1, 3, _RB, D), lambda e, r: (e, 0, r, 0)),
        pl.BlockSpec((3 * D, D), lambda e, r: (0, 0)),
        pl.BlockSpec((1, D), lambda e, r: (0, 0)),
    ],
    out_specs=pl.BlockSpec((1, _RB, D), lambda e, r: (e, r, 0)),
    out_shape=jax.ShapeDtypeStruct((2, B, D), jnp.float32),
)


def kernel(table, fc_w, fc_b,
           evtq_s_ids, evtq_s_lengths, evtq_p_ids, evtq_p_lengths,
           evtq_o_ids, evtq_o_lengths,
           evtk_s_ids, evtk_s_lengths, evtk_p_ids, evtk_p_lengths,
           evtk_o_ids, evtk_o_lengths):
    ids_all = jnp.stack([evtq_s_ids, evtq_p_ids, evtq_o_ids,
                         evtk_s_ids, evtk_p_ids, evtk_o_ids])
    lens_all = jnp.stack([evtq_s_lengths, evtq_p_lengths, evtq_o_lengths,
                          evtk_s_lengths, evtk_p_lengths, evtk_o_lengths])
    pos = jnp.arange(L, dtype=lens_all.dtype)
    idsm = jnp.where(pos[None, None, :] < lens_all[:, :, None],
                     ids_all, ids_all[:, :, :1]).astype(jnp.int32)
    coef = (lens_all.astype(jnp.float32) - L) * (1.0 / L)

    pooled = _pool(table, idsm.reshape(-1), coef.reshape(-1))
    out2 = _fc(pooled.reshape(2, 3, B, D), fc_w, fc_b.reshape(1, D))
    return out2[0], out2[1]


# speedup vs baseline: 2.8208x; 1.9334x over previous
import functools

import jax
import jax.numpy as jnp
from jax import lax
from jax.experimental import pallas as pl
from jax.experimental.pallas import tpu as pltpu
from jax.experimental.pallas import tpu_sc as plsc

D = 128
B = 4096
L = 20
NSEQ = 6
R = NSEQ * B
NC = 2
NS = 16
NW = NC * NS
RW = R // NW
G = 6
NIDX = G * L
STEPS = RW // G
NDBLK = D // 16


NBUF = 4
OSTEP = NBUF * G


def _pool_body(table_hbm, ids_hbm, coef_hbm, out_hbm,
               ids_v, coef_v, rows0, rows1, rows2, rows3, out_v,
               g0, g1, g2, g3, osem):
    wid = lax.axis_index("s") * NC + lax.axis_index("c")
    base_row = wid * RW
    rbufs = (rows0, rows1, rows2, rows3)
    gsems = (g0, g1, g2, g3)

    pltpu.sync_copy(ids_hbm.at[pl.ds(base_row * L, RW * L)], ids_v)
    pltpu.sync_copy(coef_hbm.at[pl.ds(base_row, RW)], coef_v.at[pl.ds(0, RW)])

    def start_gather(s, rows_buf, sem):
        idx = ids_v.at[pl.ds(s * NIDX, NIDX)]
        pltpu.async_copy(table_hbm.at[idx], rows_buf, sem)

    def wait_gather(rows_buf, sem):
        pltpu.make_async_copy(table_hbm.at[pl.ds(0, NIDX)], rows_buf, sem).wait()

    def start_out(so):
        pltpu.async_copy(
            out_v, out_hbm.at[pl.ds((base_row + so * OSTEP) * D, OSTEP * D)],
            osem)

    def wait_out():
        pltpu.make_async_copy(
            out_v, out_hbm.at[pl.ds(0, OSTEP * D)], osem).wait()

    def compute(s, b, rows_buf):
        cvec = coef_v[pl.ds(s * G, 16)]
        for i in range(G):
            c = cvec[i]

            def dbody(dblk, _):
                sl = pl.ds(dblk * 16, 16)
                accs = [None] * 4
                e0 = None
                for j in range(L):
                    v = rows_buf[i * L + j, sl]
                    if j == 0:
                        e0 = v
                    k = j % 4
                    accs[k] = v if accs[k] is None else accs[k] + v
                acc = (accs[0] + accs[1]) + (accs[2] + accs[3])
                out_v[pl.ds((b * G + i) * D + dblk * 16, 16)] = (
                    acc * (1.0 / L) + c * e0)
                return _

            lax.fori_loop(0, NDBLK, dbody, 0)

    for b in range(NBUF):
        start_gather(b, rbufs[b], gsems[b])

    def body(so, carry):
        @pl.when(so >= 1)
        def _():
            wait_out()

        for b in range(NBUF):
            s = so * NBUF + b
            wait_gather(rbufs[b], gsems[b])
            compute(s, b, rbufs[b])

            @pl.when(s + NBUF < STEPS)
            def _():
                start_gather(s + NBUF, rbufs[b], gsems[b])

        start_out(so)
        return carry

    lax.fori_loop(0, STEPS // NBUF, body, 0)
    wait_out()


@functools.partial(
    pl.kernel,
    mesh=plsc.VectorSubcoreMesh(core_axis_name="c", subcore_axis_name="s"),
    out_type=jax.ShapeDtypeStruct((R * D,), jnp.float32),
    scratch_types=[
        pltpu.VMEM((RW * L,), jnp.int32),
        pltpu.VMEM((RW + 16,), jnp.float32),
        pltpu.VMEM((NIDX, D), jnp.float32),
        pltpu.VMEM((NIDX, D), jnp.float32),
        pltpu.VMEM((NIDX, D), jnp.float32),
        pltpu.VMEM((NIDX, D), jnp.float32),
        pltpu.VMEM((OSTEP * D,), jnp.float32),
        pltpu.SemaphoreType.DMA,
        pltpu.SemaphoreType.DMA,
        pltpu.SemaphoreType.DMA,
        pltpu.SemaphoreType.DMA,
        pltpu.SemaphoreType.DMA,
    ],
)
def _pool(table_hbm, ids_hbm, coef_hbm, out_hbm, *rest):
    _pool_body(table_hbm, ids_hbm, coef_hbm, out_hbm, *rest)


def _fc_body(x_ref, w_ref, b_ref, o_ref):
    w = w_ref[...]
    acc = b_ref[0][None, :].astype(jnp.float32)
    for p in range(3):
        acc = acc + jax.lax.dot_general(
            x_ref[0, p], w[p * D:(p + 1) * D, :],
            (((1,), (0,)), ((), ())),
            preferred_element_type=jnp.float32,
            precision=jax.lax.Precision.HIGHEST,
        )
    o_ref[0] = acc


_RB = 512

_fc = pl.pallas_call(
    _fc_body,
    grid=(2, B // _RB),
    in_specs=[
        pl.BlockSpec((1, 3, _RB, D), lambda e, r: (e, 0, r, 0)),
        pl.BlockSpec((3 * D, D), lambda e, r: (0, 0)),
        pl.BlockSpec((1, D), lambda e, r: (0, 0)),
    ],
    out_specs=pl.BlockSpec((1, _RB, D), lambda e, r: (e, r, 0)),
    out_shape=jax.ShapeDtypeStruct((2, B, D), jnp.float32),
)


def kernel(table, fc_w, fc_b,
           evtq_s_ids, evtq_s_lengths, evtq_p_ids, evtq_p_lengths,
           evtq_o_ids, evtq_o_lengths,
           evtk_s_ids, evtk_s_lengths, evtk_p_ids, evtk_p_lengths,
           evtk_o_ids, evtk_o_lengths):
    ids_all = jnp.stack([evtq_s_ids, evtq_p_ids, evtq_o_ids,
                         evtk_s_ids, evtk_p_ids, evtk_o_ids])
    lens_all = jnp.stack([evtq_s_lengths, evtq_p_lengths, evtq_o_lengths,
                          evtk_s_lengths, evtk_p_lengths, evtk_o_lengths])
    pos = jnp.arange(L, dtype=lens_all.dtype)
    idsm = jnp.where(pos[None, None, :] < lens_all[:, :, None],
                     ids_all, ids_all[:, :, :1]).astype(jnp.int32)
    coef = (lens_all.astype(jnp.float32) - L) * (1.0 / L)

    pooled = _pool(table, idsm.reshape(-1), coef.reshape(-1))
    out2 = _fc(pooled.reshape(2, 3, B, D), fc_w, fc_b.reshape(1, D))
    return out2[0], out2[1]
